# baseline (speedup 1.0000x reference)
import functools

import jax
import jax.numpy as jnp
from jax import lax
from jax.experimental import pallas as pl
from jax.experimental.pallas import tpu as pltpu
from jax.experimental.pallas import tpu_sc as plsc

T, H, I, E, K = 2048, 1024, 2048, 8, 2
TK = T * K
BLK = 256
NROWS = TK + E * BLK
NB = NROWS // BLK


def _routing(expert_affinities, expert_index):
    e_flat = expert_index.reshape(TK)
    onehot = (e_flat[:, None] == jnp.arange(E, dtype=jnp.int32)[None, :])
    prefix = jnp.cumsum(onehot.astype(jnp.int32), axis=0)
    counts = prefix[-1]
    rank = jnp.take_along_axis(prefix, e_flat[:, None], axis=1)[:, 0] - 1
    padded = ((counts + BLK - 1) // BLK) * BLK
    cum_padded = jnp.cumsum(padded)
    start_pad = cum_padded - padded
    dest = start_pad[e_flat] + rank

    sel_aff = jnp.take_along_axis(expert_affinities, expert_index, axis=1)
    sel_aff = sel_aff / jnp.sum(sel_aff, axis=1, keepdims=True)
    w_flat = sel_aff.reshape(TK)

    tok = jnp.arange(TK, dtype=jnp.int32) // K
    row_token = jnp.zeros((NROWS,), jnp.int32).at[dest].set(tok)
    row_weight = jnp.zeros((NROWS,), jnp.float32).at[dest].set(w_flat)
    pair_pos = dest.reshape(T, K)
    block_expert = jnp.clip(
        jnp.searchsorted(cum_padded, jnp.arange(NB, dtype=jnp.int32) * BLK,
                         side="right"),
        0, E - 1).astype(jnp.int32)
    num_blocks = (cum_padded[-1] // BLK).astype(jnp.int32)
    return row_token, row_weight, block_expert, pair_pos, num_blocks


def _mlp_block(be_ref, x_ref, wg_ref, wu_ref, wd_ref, wt_ref, o_ref):
    @pl.when(pl.program_id(0) < be_ref[NB])
    def _():
        x = x_ref[...].astype(jnp.bfloat16)
        gate = jnp.dot(x, wg_ref[0].astype(jnp.bfloat16),
                       preferred_element_type=jnp.float32)
        up = jnp.dot(x, wu_ref[0].astype(jnp.bfloat16),
                     preferred_element_type=jnp.float32)
        act = (gate * jax.nn.sigmoid(gate) * up).astype(jnp.bfloat16)
        out = jnp.dot(act, wd_ref[0].astype(jnp.bfloat16),
                      preferred_element_type=jnp.float32)
        o_ref[...] = out * wt_ref[...]


def _grouped_mlp(xs, w_gate, w_up, w_down, row_weight, scalars):
    grid_spec = pltpu.PrefetchScalarGridSpec(
        num_scalar_prefetch=1,
        grid=(NB,),
        in_specs=[
            pl.BlockSpec((BLK, H), lambda i, be: (i, 0)),
            pl.BlockSpec((1, H, I), lambda i, be: (be[i], 0, 0)),
            pl.BlockSpec((1, H, I), lambda i, be: (be[i], 0, 0)),
            pl.BlockSpec((1, I, H), lambda i, be: (be[i], 0, 0)),
            pl.BlockSpec((BLK, 1), lambda i, be: (i, 0)),
        ],
        out_specs=pl.BlockSpec((BLK, H), lambda i, be: (i, 0)),
    )
    return pl.pallas_call(
        _mlp_block,
        grid_spec=grid_spec,
        out_shape=jax.ShapeDtypeStruct((NROWS, H), jnp.float32),
    )(scalars, xs, w_gate, w_up, w_down, row_weight.reshape(NROWS, 1))


_SC_MESH = plsc.VectorSubcoreMesh(core_axis_name="c", subcore_axis_name="s")
_NW = 32
_DISP_RPW = NROWS // _NW
_DISP_CH = _DISP_RPW // 2
_COMB_TPW = T // _NW
_COMB_CH = _COMB_TPW // 2


def _sc_wid():
    return lax.axis_index("s") * 2 + lax.axis_index("c")


@functools.partial(
    pl.kernel, mesh=_SC_MESH,
    out_type=jax.ShapeDtypeStruct((NROWS, H), jnp.float32),
    scratch_types=[
        pltpu.VMEM((_DISP_CH,), jnp.int32),
        pltpu.VMEM((_DISP_CH,), jnp.int32),
        pltpu.VMEM((_DISP_CH, H), jnp.float32),
        pltpu.SemaphoreType.DMA,
    ],
)
def _sc_dispatch(hidden_hbm, tok_hbm, xs_hbm, idx_a, idx_b, rows_v, sem):
    base = _sc_wid() * _DISP_RPW
    pltpu.sync_copy(tok_hbm.at[pl.ds(base, _DISP_CH)], idx_a)
    pltpu.sync_copy(tok_hbm.at[pl.ds(base + _DISP_CH, _DISP_CH)], idx_b)
    pltpu.async_copy(hidden_hbm.at[idx_a], rows_v, sem).wait()
    pltpu.sync_copy(rows_v, xs_hbm.at[pl.ds(base, _DISP_CH)])
    pltpu.async_copy(hidden_hbm.at[idx_b], rows_v, sem).wait()
    pltpu.sync_copy(rows_v, xs_hbm.at[pl.ds(base + _DISP_CH, _DISP_CH)])


@functools.partial(
    pl.kernel, mesh=_SC_MESH,
    out_type=jax.ShapeDtypeStruct((T, H), jnp.float32),
    scratch_types=[
        pltpu.VMEM((_COMB_CH,), jnp.int32),
        pltpu.VMEM((_COMB_CH,), jnp.int32),
        pltpu.VMEM((_COMB_CH, H), jnp.float32),
        pltpu.VMEM((_COMB_CH, H), jnp.float32),
        pltpu.SemaphoreType.DMA,
        pltpu.SemaphoreType.DMA,
    ],
)
def _sc_combine(ys_hbm, pos0_hbm, pos1_hbm, out_hbm,
                idx0, idx1, buf0, buf1, sem0, sem1):
    wbase = _sc_wid() * _COMB_TPW
    for c in range(2):
        tb = wbase + c * _COMB_CH
        pltpu.sync_copy(pos0_hbm.at[pl.ds(tb, _COMB_CH)], idx0)
        pltpu.sync_copy(pos1_hbm.at[pl.ds(tb, _COMB_CH)], idx1)
        cp0 = pltpu.async_copy(ys_hbm.at[idx0], buf0, sem0)
        cp1 = pltpu.async_copy(ys_hbm.at[idx1], buf1, sem1)
        cp0.wait()
        cp1.wait()

        def _add(i, _):
            r = i >> 4
            c0 = pl.multiple_of((i & 15) << 6, 64)
            for u in range(4):
                sl = pl.ds(c0 + u * 16, 16)
                buf0[r, sl] = buf0[r, sl] + buf1[r, sl]
            return 0

        lax.fori_loop(0, _COMB_CH * 16, _add, 0, unroll=False)
        pltpu.sync_copy(buf0, out_hbm.at[pl.ds(tb, _COMB_CH)])


def kernel(hidden_states, expert_affinities, expert_index, w_gate, w_up,
           w_down, seq_len):
    row_token, row_weight, block_expert, pair_pos, num_blocks = _routing(
        expert_affinities, expert_index)
    scalars = jnp.concatenate([block_expert, num_blocks[None]])
    xs = _sc_dispatch(hidden_states, row_token)
    ys = _grouped_mlp(xs, w_gate, w_up, w_down, row_weight, scalars)
    out = _sc_combine(ys, pair_pos[:, 0].ravel(), pair_pos[:, 1].ravel())
    return out


# speedup vs baseline: 1.4198x; 1.4198x over previous
import functools

import jax
import jax.numpy as jnp
from jax import lax
from jax.experimental import pallas as pl
from jax.experimental.pallas import tpu as pltpu
from jax.experimental.pallas import tpu_sc as plsc

T, H, I, E, K = 2048, 1024, 2048, 8, 2
TK = T * K
BLK = 128
NROWS = TK + E * BLK
NB = NROWS // BLK


def _routing(expert_affinities, expert_index):
    e_flat = expert_index.reshape(TK)
    onehot = (e_flat[:, None] == jnp.arange(E, dtype=jnp.int32)[None, :])
    prefix = jnp.cumsum(onehot.astype(jnp.int32), axis=0)
    counts = prefix[-1]
    rank = jnp.take_along_axis(prefix, e_flat[:, None], axis=1)[:, 0] - 1
    padded = ((counts + BLK - 1) // BLK) * BLK
    cum_padded = jnp.cumsum(padded)
    start_pad = cum_padded - padded
    dest = start_pad[e_flat] + rank

    sel_aff = jnp.take_along_axis(expert_affinities, expert_index, axis=1)
    sel_aff = sel_aff / jnp.sum(sel_aff, axis=1, keepdims=True)

    pair_pos = dest.reshape(T, K)
    pos0, pos1 = pair_pos[:, 0], pair_pos[:, 1]
    w0x = jnp.broadcast_to(sel_aff[:, 0:1], (T, 16))
    w1x = jnp.broadcast_to(sel_aff[:, 1:2], (T, 16))
    block_expert = jnp.clip(
        jnp.searchsorted(cum_padded, jnp.arange(NB, dtype=jnp.int32) * BLK,
                         side="right"),
        0, E - 1).astype(jnp.int32)
    num_blocks = (cum_padded[-1] // BLK).astype(jnp.int32)
    scalars = jnp.concatenate([block_expert, num_blocks[None]])
    return pos0, pos1, w0x, w1x, scalars


def _mlp_block(be_ref, x_ref, wg_ref, wu_ref, wd_ref, o_ref):
    @pl.when(pl.program_id(0) < be_ref[NB])
    def _():
        x = x_ref[...].astype(jnp.bfloat16)
        gate = jnp.dot(x, wg_ref[0].astype(jnp.bfloat16),
                       preferred_element_type=jnp.float32)
        up = jnp.dot(x, wu_ref[0].astype(jnp.bfloat16),
                     preferred_element_type=jnp.float32)
        act = (gate * jax.nn.sigmoid(gate) * up).astype(jnp.bfloat16)
        out = jnp.dot(act, wd_ref[0].astype(jnp.bfloat16),
                      preferred_element_type=jnp.float32)
        o_ref[...] = out


def _grouped_mlp(xs, w_gate, w_up, w_down, scalars):
    grid_spec = pltpu.PrefetchScalarGridSpec(
        num_scalar_prefetch=1,
        grid=(NB,),
        in_specs=[
            pl.BlockSpec((BLK, H), lambda i, be: (i, 0)),
            pl.BlockSpec((1, H, I), lambda i, be: (be[i], 0, 0)),
            pl.BlockSpec((1, H, I), lambda i, be: (be[i], 0, 0)),
            pl.BlockSpec((1, I, H), lambda i, be: (be[i], 0, 0)),
        ],
        out_specs=pl.BlockSpec((BLK, H), lambda i, be: (i, 0)),
    )
    return pl.pallas_call(
        _mlp_block,
        grid_spec=grid_spec,
        out_shape=jax.ShapeDtypeStruct((NROWS, H), jnp.float32),
    )(scalars, xs, w_gate, w_up, w_down)


_SC_MESH = plsc.VectorSubcoreMesh(core_axis_name="c", subcore_axis_name="s")
_NW = 32
_DISP_TPW = T // _NW
_COMB_TPW = T // _NW
_COMB_CH = _COMB_TPW // 2


def _sc_wid():
    return lax.axis_index("s") * 2 + lax.axis_index("c")


@functools.partial(
    pl.kernel, mesh=_SC_MESH,
    out_type=jax.ShapeDtypeStruct((NROWS, H), jnp.float32),
    scratch_types=[
        pltpu.VMEM((_DISP_TPW,), jnp.int32),
        pltpu.VMEM((_DISP_TPW,), jnp.int32),
        pltpu.VMEM((_DISP_TPW, H), jnp.float32),
        pltpu.SemaphoreType.DMA,
        pltpu.SemaphoreType.DMA,
    ],
)
def _sc_dispatch(hidden_hbm, pos0_hbm, pos1_hbm, xs_hbm,
                 idx0, idx1, rows_v, sem0, sem1):
    tbase = _sc_wid() * _DISP_TPW
    pltpu.sync_copy(pos0_hbm.at[pl.ds(tbase, _DISP_TPW)], idx0)
    pltpu.sync_copy(pos1_hbm.at[pl.ds(tbase, _DISP_TPW)], idx1)
    pltpu.sync_copy(hidden_hbm.at[pl.ds(tbase, _DISP_TPW)], rows_v)
    cp0 = pltpu.async_copy(rows_v, xs_hbm.at[idx0], sem0)
    cp1 = pltpu.async_copy(rows_v, xs_hbm.at[idx1], sem1)
    cp0.wait()
    cp1.wait()


@functools.partial(
    pl.kernel, mesh=_SC_MESH,
    out_type=jax.ShapeDtypeStruct((T, H), jnp.float32),
    scratch_types=[
        pltpu.VMEM((_COMB_CH,), jnp.int32),
        pltpu.VMEM((_COMB_CH,), jnp.int32),
        pltpu.VMEM((_COMB_CH, 16), jnp.float32),
        pltpu.VMEM((_COMB_CH, 16), jnp.float32),
        pltpu.VMEM((_COMB_CH, H), jnp.float32),
        pltpu.VMEM((_COMB_CH, H), jnp.float32),
        pltpu.SemaphoreType.DMA,
        pltpu.SemaphoreType.DMA,
    ],
)
def _sc_combine(ys_hbm, pos0_hbm, pos1_hbm, w0_hbm, w1_hbm, out_hbm,
                idx0, idx1, wv0, wv1, buf0, buf1, sem0, sem1):
    wbase = _sc_wid() * _COMB_TPW
    for c in range(2):
        tb = wbase + c * _COMB_CH
        pltpu.sync_copy(pos0_hbm.at[pl.ds(tb, _COMB_CH)], idx0)
        pltpu.sync_copy(pos1_hbm.at[pl.ds(tb, _COMB_CH)], idx1)
        pltpu.sync_copy(w0_hbm.at[pl.ds(tb, _COMB_CH)], wv0)
        pltpu.sync_copy(w1_hbm.at[pl.ds(tb, _COMB_CH)], wv1)
        cp0 = pltpu.async_copy(ys_hbm.at[idx0], buf0, sem0)
        cp1 = pltpu.async_copy(ys_hbm.at[idx1], buf1, sem1)
        cp0.wait()
        cp1.wait()

        def _add(i, _):
            r = i >> 4
            c0 = pl.multiple_of((i & 15) << 6, 64)
            a = wv0[r, :]
            b = wv1[r, :]
            for u in range(4):
                sl = pl.ds(c0 + u * 16, 16)
                buf0[r, sl] = buf0[r, sl] * a + buf1[r, sl] * b
            return 0

        lax.fori_loop(0, _COMB_CH * 16, _add, 0, unroll=False)
        pltpu.sync_copy(buf0, out_hbm.at[pl.ds(tb, _COMB_CH)])


def kernel(hidden_states, expert_affinities, expert_index, w_gate, w_up,
           w_down, seq_len):
    pos0, pos1, w0x, w1x, scalars = _routing(expert_affinities, expert_index)
    xs = _sc_dispatch(hidden_states, pos0, pos1)
    ys = _grouped_mlp(xs, w_gate, w_up, w_down, scalars)
    out = _sc_combine(ys, pos0, pos1, w0x, w1x)
    return out


# speedup vs baseline: 1.6299x; 1.1480x over previous
import functools

import jax
import jax.numpy as jnp
from jax import lax
from jax.experimental import pallas as pl
from jax.experimental.pallas import tpu as pltpu
from jax.experimental.pallas import tpu_sc as plsc

T, H, I, E, K = 2048, 1024, 2048, 8, 2
TK = T * K
BLK = 128
NROWS = TK + E * BLK
NB = NROWS // BLK


RR = 32
RC = TK // RR


def _routing_block(ei_ref, eio_ref, aff_ref, pos_ref, w0_ref, w1_ref,
                   sc_ref):
    ei2 = ei_ref[...]
    kk = lax.broadcasted_iota(jnp.int32, (RC, RC), 0)
    jj = lax.broadcasted_iota(jnp.int32, (RC, RC), 1)
    L = (kk <= jj).astype(jnp.float32)
    k2 = lax.broadcasted_iota(jnp.int32, (RR, RR), 0)
    j2 = lax.broadcasted_iota(jnp.int32, (RR, RR), 1)
    Ut = (j2 < k2).astype(jnp.float32)

    rank = jnp.zeros((RR, RC), jnp.float32)
    counts = []
    ohs = []
    for e in range(E):
        ohe = (ei2 == e).astype(jnp.float32)
        pre = jnp.dot(ohe, L, preferred_element_type=jnp.float32)
        rs = pre[:, RC - 1:RC]
        offs = jnp.dot(Ut, rs, preferred_element_type=jnp.float32)
        prefix = pre + offs
        counts.append(prefix[RR - 1:RR, RC - 1:RC])
        rank = rank + ohe * prefix
        ohs.append(ohe)
    rank = rank - 1.0

    cnt = jnp.concatenate(counts, axis=1)
    padded = jnp.ceil(cnt * (1.0 / BLK)) * BLK
    k3 = lax.broadcasted_iota(jnp.int32, (E, E), 0)
    j3 = lax.broadcasted_iota(jnp.int32, (E, E), 1)
    LE = (k3 <= j3).astype(jnp.float32)
    cum_padded = jnp.dot(padded, LE,
                         preferred_element_type=jnp.float32)
    start_pad = cum_padded - padded
    dest = rank
    for e in range(E):
        dest = dest + ohs[e] * start_pad[0:1, e:e + 1]
    pos_ref[...] = dest.astype(jnp.int32)

    aff = aff_ref[...]
    ti = lax.broadcasted_iota(jnp.int32, (T, E), 1)
    m0 = (eio_ref[:, 0:1] == ti).astype(jnp.float32)
    m1 = (eio_ref[:, 1:2] == ti).astype(jnp.float32)
    a0 = jnp.sum(aff * m0, axis=1, keepdims=True)
    a1 = jnp.sum(aff * m1, axis=1, keepdims=True)
    inv = 1.0 / (a0 + a1)
    w0_ref[...] = jnp.broadcast_to(a0 * inv, (T, 16))
    w1_ref[...] = jnp.broadcast_to(a1 * inv, (T, 16))

    bpos = lax.broadcasted_iota(jnp.int32, (NB, 1), 0).astype(
        jnp.float32) * BLK
    ge = (bpos >= cum_padded).astype(jnp.int32)
    be = jnp.sum(ge, axis=1, keepdims=True)
    be = jnp.minimum(be, E - 1)
    nb_used = (cum_padded[0:1, E - 1:E] * (1.0 / BLK)).astype(jnp.int32)
    sc_ref[...] = jnp.concatenate(
        [be, jnp.broadcast_to(nb_used, (8, 1))], axis=0)


def _routing(expert_affinities, expert_index):
    ei_cm = jnp.concatenate(
        [expert_index[:, 0], expert_index[:, 1]]).reshape(RR, RC)
    pos, w0x, w1x, sc2 = pl.pallas_call(
        _routing_block,
        out_shape=(
            jax.ShapeDtypeStruct((RR, RC), jnp.int32),
            jax.ShapeDtypeStruct((T, 16), jnp.float32),
            jax.ShapeDtypeStruct((T, 16), jnp.float32),
            jax.ShapeDtypeStruct((NB + 8, 1), jnp.int32),
        ),
    )(ei_cm, expert_index, expert_affinities)
    pos_flat = pos.reshape(TK)
    return (pos_flat[:T], pos_flat[T:], w0x, w1x, sc2[:NB + 1, 0])


def _mlp_block(be_ref, x_ref, wg_ref, wu_ref, wd_ref, o_ref):
    @pl.when(pl.program_id(0) < be_ref[NB])
    def _():
        x = x_ref[...].astype(jnp.bfloat16)
        gate = jnp.dot(x, wg_ref[0].astype(jnp.bfloat16),
                       preferred_element_type=jnp.float32)
        up = jnp.dot(x, wu_ref[0].astype(jnp.bfloat16),
                     preferred_element_type=jnp.float32)
        act = (gate * jax.nn.sigmoid(gate) * up).astype(jnp.bfloat16)
        out = jnp.dot(act, wd_ref[0].astype(jnp.bfloat16),
                      preferred_element_type=jnp.float32)
        o_ref[...] = out


def _grouped_mlp(xs, w_gate, w_up, w_down, scalars):
    grid_spec = pltpu.PrefetchScalarGridSpec(
        num_scalar_prefetch=1,
        grid=(NB,),
        in_specs=[
            pl.BlockSpec((BLK, H), lambda i, be: (i, 0)),
            pl.BlockSpec((1, H, I), lambda i, be: (be[i], 0, 0)),
            pl.BlockSpec((1, H, I), lambda i, be: (be[i], 0, 0)),
            pl.BlockSpec((1, I, H), lambda i, be: (be[i], 0, 0)),
        ],
        out_specs=pl.BlockSpec((BLK, H), lambda i, be: (i, 0)),
    )
    return pl.pallas_call(
        _mlp_block,
        grid_spec=grid_spec,
        out_shape=jax.ShapeDtypeStruct((NROWS, H), jnp.float32),
    )(scalars, xs, w_gate, w_up, w_down)


_NW = 32
_DISP_TPW = T // _NW
_COMB_TPW = T // _NW
_COMB_CH = _COMB_TPW // 2


def _sc_wid():
    return lax.axis_index("s") * 2 + lax.axis_index("c")


@functools.cache
def _sc_kernels():
    mesh = plsc.VectorSubcoreMesh(core_axis_name="c", subcore_axis_name="s")

    @functools.partial(
        pl.kernel, mesh=mesh,
        out_type=jax.ShapeDtypeStruct((NROWS, H), jnp.float32),
        scratch_types=[
            pltpu.VMEM((_DISP_TPW,), jnp.int32),
            pltpu.VMEM((_DISP_TPW,), jnp.int32),
            pltpu.VMEM((_DISP_TPW, H), jnp.float32),
            pltpu.SemaphoreType.DMA,
            pltpu.SemaphoreType.DMA,
        ],
    )
    def _sc_dispatch(hidden_hbm, pos0_hbm, pos1_hbm, xs_hbm,
                     idx0, idx1, rows_v, sem0, sem1):
        tbase = _sc_wid() * _DISP_TPW
        pltpu.sync_copy(pos0_hbm.at[pl.ds(tbase, _DISP_TPW)], idx0)
        pltpu.sync_copy(pos1_hbm.at[pl.ds(tbase, _DISP_TPW)], idx1)
        pltpu.sync_copy(hidden_hbm.at[pl.ds(tbase, _DISP_TPW)], rows_v)
        cp0 = pltpu.async_copy(rows_v, xs_hbm.at[idx0], sem0)
        cp1 = pltpu.async_copy(rows_v, xs_hbm.at[idx1], sem1)
        cp0.wait()
        cp1.wait()

    @functools.partial(
        pl.kernel, mesh=mesh,
        out_type=jax.ShapeDtypeStruct((T, H), jnp.float32),
        scratch_types=[
            pltpu.VMEM((_COMB_CH,), jnp.int32),
            pltpu.VMEM((_COMB_CH,), jnp.int32),
            pltpu.VMEM((_COMB_CH, 16), jnp.float32),
            pltpu.VMEM((_COMB_CH, 16), jnp.float32),
            pltpu.VMEM((_COMB_CH, H), jnp.float32),
            pltpu.VMEM((_COMB_CH, H), jnp.float32),
            pltpu.SemaphoreType.DMA,
            pltpu.SemaphoreType.DMA,
        ],
    )
    def _sc_combine(ys_hbm, pos0_hbm, pos1_hbm, w0_hbm, w1_hbm, out_hbm,
                    idx0, idx1, wv0, wv1, buf0, buf1, sem0, sem1):
        wbase = _sc_wid() * _COMB_TPW
        for c in range(2):
            tb = wbase + c * _COMB_CH
            pltpu.sync_copy(pos0_hbm.at[pl.ds(tb, _COMB_CH)], idx0)
            pltpu.sync_copy(pos1_hbm.at[pl.ds(tb, _COMB_CH)], idx1)
            pltpu.sync_copy(w0_hbm.at[pl.ds(tb, _COMB_CH)], wv0)
            pltpu.sync_copy(w1_hbm.at[pl.ds(tb, _COMB_CH)], wv1)
            cp0 = pltpu.async_copy(ys_hbm.at[idx0], buf0, sem0)
            cp1 = pltpu.async_copy(ys_hbm.at[idx1], buf1, sem1)
            cp0.wait()
            cp1.wait()

            def _add(r, _):
                a = wv0[r, :]
                b = wv1[r, :]
                for u in range(H // 16):
                    sl = pl.ds(u * 16, 16)
                    buf0[r, sl] = buf0[r, sl] * a + buf1[r, sl] * b
                return 0

            lax.fori_loop(0, _COMB_CH, _add, 0, unroll=False)
            pltpu.sync_copy(buf0, out_hbm.at[pl.ds(tb, _COMB_CH)])

    return _sc_dispatch, _sc_combine


def kernel(hidden_states, expert_affinities, expert_index, w_gate, w_up,
           w_down, seq_len):
    sc_dispatch, sc_combine = _sc_kernels()
    pos0, pos1, w0x, w1x, scalars = _routing(expert_affinities, expert_index)
    xs = sc_dispatch(hidden_states, pos0, pos1)
    ys = _grouped_mlp(xs, w_gate, w_up, w_down, scalars)
    out = sc_combine(ys, pos0, pos1, w0x, w1x)
    return out


# speedup vs baseline: 1.6491x; 1.0118x over previous
import functools

import jax
import jax.numpy as jnp
from jax import lax
from jax.experimental import pallas as pl
from jax.experimental.pallas import tpu as pltpu
from jax.experimental.pallas import tpu_sc as plsc

T, H, I, E, K = 2048, 1024, 2048, 8, 2
TK = T * K
BLK = 128
NROWS = TK + E * BLK
NB = NROWS // BLK


RR = 32
RC = TK // RR


def _routing_block(ei_ref, eio_ref, aff_ref, pos0_ref, pos1_ref, w0_ref,
                   w1_ref, sc_ref):
    ei2 = ei_ref[...]
    kk = lax.broadcasted_iota(jnp.int32, (RC, RC), 0)
    jj = lax.broadcasted_iota(jnp.int32, (RC, RC), 1)
    L = (kk <= jj).astype(jnp.float32)
    k2 = lax.broadcasted_iota(jnp.int32, (RR, RR), 0)
    j2 = lax.broadcasted_iota(jnp.int32, (RR, RR), 1)
    Ut = (j2 < k2).astype(jnp.float32)

    rank = jnp.zeros((RR, RC), jnp.float32)
    counts = []
    ohs = []
    for e in range(E):
        ohe = (ei2 == e).astype(jnp.float32)
        pre = jnp.dot(ohe, L, preferred_element_type=jnp.float32)
        rs = pre[:, RC - 1:RC]
        offs = jnp.dot(Ut, rs, preferred_element_type=jnp.float32)
        prefix = pre + offs
        counts.append(prefix[RR - 1:RR, RC - 1:RC])
        rank = rank + ohe * prefix
        ohs.append(ohe)
    rank = rank - 1.0

    cnt = jnp.concatenate(counts, axis=1)
    padded = jnp.ceil(cnt * (1.0 / BLK)) * BLK
    k3 = lax.broadcasted_iota(jnp.int32, (E, E), 0)
    j3 = lax.broadcasted_iota(jnp.int32, (E, E), 1)
    LE = (k3 <= j3).astype(jnp.float32)
    cum_padded = jnp.dot(padded, LE,
                         preferred_element_type=jnp.float32)
    start_pad = cum_padded - padded
    dest = rank
    for e in range(E):
        dest = dest + ohs[e] * start_pad[0:1, e:e + 1]
    dest_i = dest.astype(jnp.int32)
    pos0_ref[...] = dest_i[0:RR // 2, :]
    pos1_ref[...] = dest_i[RR // 2:RR, :]

    aff = aff_ref[...]
    ti = lax.broadcasted_iota(jnp.int32, (T, E), 1)
    m0 = (eio_ref[:, 0:1] == ti).astype(jnp.float32)
    m1 = (eio_ref[:, 1:2] == ti).astype(jnp.float32)
    a0 = jnp.sum(aff * m0, axis=1, keepdims=True)
    a1 = jnp.sum(aff * m1, axis=1, keepdims=True)
    inv = 1.0 / (a0 + a1)
    w0_ref[...] = jnp.broadcast_to(a0 * inv, (T, 16))
    w1_ref[...] = jnp.broadcast_to(a1 * inv, (T, 16))

    bpos = lax.broadcasted_iota(jnp.int32, (NB, 1), 0).astype(
        jnp.float32) * BLK
    ge = (bpos >= cum_padded).astype(jnp.int32)
    be = jnp.sum(ge, axis=1, keepdims=True)
    be = jnp.minimum(be, E - 1)
    nb_used = (cum_padded[0:1, E - 1:E] * (1.0 / BLK)).astype(jnp.int32)
    sc_ref[...] = jnp.concatenate(
        [be, jnp.broadcast_to(nb_used, (8, 1))], axis=0)


def _routing(expert_affinities, expert_index):
    ei_cm = jnp.concatenate(
        [expert_index[:, 0], expert_index[:, 1]]).reshape(RR, RC)
    return pl.pallas_call(
        _routing_block,
        out_shape=(
            jax.ShapeDtypeStruct((RR // 2, RC), jnp.int32),
            jax.ShapeDtypeStruct((RR // 2, RC), jnp.int32),
            jax.ShapeDtypeStruct((T, 16), jnp.float32),
            jax.ShapeDtypeStruct((T, 16), jnp.float32),
            jax.ShapeDtypeStruct((NB + 8, 1), jnp.int32),
        ),
    )(ei_cm, expert_index, expert_affinities)


def _mlp_block(be_ref, x_ref, wg_ref, wu_ref, wd_ref, o_ref):
    @pl.when(pl.program_id(0) < be_ref[NB, 0])
    def _():
        x = x_ref[...].astype(jnp.bfloat16)
        gate = jnp.dot(x, wg_ref[0].astype(jnp.bfloat16),
                       preferred_element_type=jnp.float32)
        up = jnp.dot(x, wu_ref[0].astype(jnp.bfloat16),
                     preferred_element_type=jnp.float32)
        act = (gate * jax.nn.sigmoid(gate) * up).astype(jnp.bfloat16)
        out = jnp.dot(act, wd_ref[0].astype(jnp.bfloat16),
                      preferred_element_type=jnp.float32)
        o_ref[...] = out


def _grouped_mlp(xs, w_gate, w_up, w_down, scalars):
    grid_spec = pltpu.PrefetchScalarGridSpec(
        num_scalar_prefetch=1,
        grid=(NB,),
        in_specs=[
            pl.BlockSpec((BLK, H), lambda i, be: (i, 0)),
            pl.BlockSpec((1, H, I), lambda i, be: (be[i, 0], 0, 0)),
            pl.BlockSpec((1, H, I), lambda i, be: (be[i, 0], 0, 0)),
            pl.BlockSpec((1, I, H), lambda i, be: (be[i, 0], 0, 0)),
        ],
        out_specs=pl.BlockSpec((BLK, H), lambda i, be: (i, 0)),
    )
    return pl.pallas_call(
        _mlp_block,
        grid_spec=grid_spec,
        out_shape=jax.ShapeDtypeStruct((NROWS, H), jnp.float32),
    )(scalars, xs, w_gate, w_up, w_down)


_NW = 32
_DISP_TPW = T // _NW
_COMB_TPW = T // _NW
_COMB_CH = _COMB_TPW // 2


def _sc_wid():
    return lax.axis_index("s") * 2 + lax.axis_index("c")


@functools.cache
def _sc_kernels():
    mesh = plsc.VectorSubcoreMesh(core_axis_name="c", subcore_axis_name="s")

    @functools.partial(
        pl.kernel, mesh=mesh,
        out_type=jax.ShapeDtypeStruct((NROWS, H), jnp.float32),
        scratch_types=[
            pltpu.VMEM((_DISP_TPW,), jnp.int32),
            pltpu.VMEM((_DISP_TPW,), jnp.int32),
            pltpu.VMEM((_DISP_TPW, H), jnp.float32),
            pltpu.SemaphoreType.DMA,
            pltpu.SemaphoreType.DMA,
        ],
    )
    def _sc_dispatch(hidden_hbm, pos0_hbm, pos1_hbm, xs_hbm,
                     idx0, idx1, rows_v, sem0, sem1):
        tbase = _sc_wid() * _DISP_TPW
        pltpu.sync_copy(pos0_hbm.at[pl.ds(tbase, _DISP_TPW)], idx0)
        pltpu.sync_copy(pos1_hbm.at[pl.ds(tbase, _DISP_TPW)], idx1)
        pltpu.sync_copy(hidden_hbm.at[pl.ds(tbase, _DISP_TPW)], rows_v)
        cp0 = pltpu.async_copy(rows_v, xs_hbm.at[idx0], sem0)
        cp1 = pltpu.async_copy(rows_v, xs_hbm.at[idx1], sem1)
        cp0.wait()
        cp1.wait()

    @functools.partial(
        pl.kernel, mesh=mesh,
        out_type=jax.ShapeDtypeStruct((T, H), jnp.float32),
        scratch_types=[
            pltpu.VMEM((_COMB_CH,), jnp.int32),
            pltpu.VMEM((_COMB_CH,), jnp.int32),
            pltpu.VMEM((_COMB_CH, 16), jnp.float32),
            pltpu.VMEM((_COMB_CH, 16), jnp.float32),
            pltpu.VMEM((_COMB_CH, H), jnp.float32),
            pltpu.VMEM((_COMB_CH, H), jnp.float32),
            pltpu.SemaphoreType.DMA,
            pltpu.SemaphoreType.DMA,
        ],
    )
    def _sc_combine(ys_hbm, pos0_hbm, pos1_hbm, w0_hbm, w1_hbm, out_hbm,
                    idx0, idx1, wv0, wv1, buf0, buf1, sem0, sem1):
        wbase = _sc_wid() * _COMB_TPW
        for c in range(2):
            tb = wbase + c * _COMB_CH
            pltpu.sync_copy(pos0_hbm.at[pl.ds(tb, _COMB_CH)], idx0)
            pltpu.sync_copy(pos1_hbm.at[pl.ds(tb, _COMB_CH)], idx1)
            pltpu.sync_copy(w0_hbm.at[pl.ds(tb, _COMB_CH)], wv0)
            pltpu.sync_copy(w1_hbm.at[pl.ds(tb, _COMB_CH)], wv1)
            cp0 = pltpu.async_copy(ys_hbm.at[idx0], buf0, sem0)
            cp1 = pltpu.async_copy(ys_hbm.at[idx1], buf1, sem1)
            cp0.wait()
            cp1.wait()

            def _add(r, _):
                a = wv0[r, :]
                b = wv1[r, :]
                for u in range(H // 16):
                    sl = pl.ds(u * 16, 16)
                    buf0[r, sl] = buf0[r, sl] * a + buf1[r, sl] * b
                return 0

            lax.fori_loop(0, _COMB_CH, _add, 0, unroll=False)
            pltpu.sync_copy(buf0, out_hbm.at[pl.ds(tb, _COMB_CH)])

    return _sc_dispatch, _sc_combine


def kernel(hidden_states, expert_affinities, expert_index, w_gate, w_up,
           w_down, seq_len):
    sc_dispatch, sc_combine = _sc_kernels()
    pos0, pos1, w0x, w1x, sc2 = _routing(expert_affinities, expert_index)
    pos0, pos1 = pos0.reshape(T), pos1.reshape(T)
    xs = sc_dispatch(hidden_states, pos0, pos1)
    ys = _grouped_mlp(xs, w_gate, w_up, w_down, sc2)
    out = sc_combine(ys, pos0, pos1, w0x, w1x)
    return out


# speedup vs baseline: 1.6493x; 1.0001x over previous
import functools

import jax
import jax.numpy as jnp
from jax import lax
from jax.experimental import pallas as pl
from jax.experimental.pallas import tpu as pltpu
from jax.experimental.pallas import tpu_sc as plsc

T, H, I, E, K = 2048, 1024, 2048, 8, 2
TK = T * K
BLK = 128
NROWS = TK + E * BLK
NB = NROWS // BLK


RR = 32
RC = TK // RR


def _routing_block(ei_ref, eio_ref, aff_ref, pos0_ref, pos1_ref, w0_ref,
                   w1_ref, sc_ref):
    ei2 = ei_ref[...]
    kk = lax.broadcasted_iota(jnp.int32, (RC, RC), 0)
    jj = lax.broadcasted_iota(jnp.int32, (RC, RC), 1)
    L = (kk <= jj).astype(jnp.float32)
    k2 = lax.broadcasted_iota(jnp.int32, (RR, RR), 0)
    j2 = lax.broadcasted_iota(jnp.int32, (RR, RR), 1)
    Ut = (j2 < k2).astype(jnp.float32)

    rank = jnp.zeros((RR, RC), jnp.float32)
    counts = []
    ohs = []
    for e in range(E):
        ohe = (ei2 == e).astype(jnp.float32)
        pre = jnp.dot(ohe, L, preferred_element_type=jnp.float32)
        rs = pre[:, RC - 1:RC]
        offs = jnp.dot(Ut, rs, preferred_element_type=jnp.float32)
        prefix = pre + offs
        counts.append(prefix[RR - 1:RR, RC - 1:RC])
        rank = rank + ohe * prefix
        ohs.append(ohe)
    rank = rank - 1.0

    cnt = jnp.concatenate(counts, axis=1)
    padded = jnp.ceil(cnt * (1.0 / BLK)) * BLK
    k3 = lax.broadcasted_iota(jnp.int32, (E, E), 0)
    j3 = lax.broadcasted_iota(jnp.int32, (E, E), 1)
    LE = (k3 <= j3).astype(jnp.float32)
    cum_padded = jnp.dot(padded, LE,
                         preferred_element_type=jnp.float32)
    start_pad = cum_padded - padded
    dest = rank
    for e in range(E):
        dest = dest + ohs[e] * start_pad[0:1, e:e + 1]
    dest_i = dest.astype(jnp.int32)
    pos0_ref[...] = dest_i[0:RR // 2, :]
    pos1_ref[...] = dest_i[RR // 2:RR, :]

    aff = aff_ref[...]
    ti = lax.broadcasted_iota(jnp.int32, (T, E), 1)
    m0 = (eio_ref[:, 0:1] == ti).astype(jnp.float32)
    m1 = (eio_ref[:, 1:2] == ti).astype(jnp.float32)
    a0 = jnp.sum(aff * m0, axis=1, keepdims=True)
    a1 = jnp.sum(aff * m1, axis=1, keepdims=True)
    inv = 1.0 / (a0 + a1)
    w0_ref[...] = jnp.broadcast_to(a0 * inv, (T, 16))
    w1_ref[...] = jnp.broadcast_to(a1 * inv, (T, 16))

    bpos = lax.broadcasted_iota(jnp.int32, (NB, 1), 0).astype(
        jnp.float32) * BLK
    ge = (bpos >= cum_padded).astype(jnp.int32)
    be = jnp.sum(ge, axis=1, keepdims=True)
    be = jnp.minimum(be, E - 1)
    nb_used = (cum_padded[0:1, E - 1:E] * (1.0 / BLK)).astype(jnp.int32)
    sc_ref[...] = jnp.concatenate(
        [be, jnp.broadcast_to(nb_used, (8, 1))], axis=0)


def _routing(expert_affinities, expert_index):
    ei_cm = jnp.concatenate(
        [expert_index[:, 0], expert_index[:, 1]]).reshape(RR, RC)
    return pl.pallas_call(
        _routing_block,
        out_shape=(
            jax.ShapeDtypeStruct((RR // 2, RC), jnp.int32),
            jax.ShapeDtypeStruct((RR // 2, RC), jnp.int32),
            jax.ShapeDtypeStruct((T, 16), jnp.float32),
            jax.ShapeDtypeStruct((T, 16), jnp.float32),
            jax.ShapeDtypeStruct((NB + 8, 1), jnp.int32),
        ),
    )(ei_cm, expert_index, expert_affinities)


def _mlp_block(be_ref, x_ref, wg_ref, wu_ref, wd_ref, o_ref):
    @pl.when(pl.program_id(0) < be_ref[NB, 0])
    def _():
        x = x_ref[...].astype(jnp.bfloat16)
        gate = jnp.dot(x, wg_ref[0].astype(jnp.bfloat16),
                       preferred_element_type=jnp.float32)
        up = jnp.dot(x, wu_ref[0].astype(jnp.bfloat16),
                     preferred_element_type=jnp.float32)
        act = (gate * jax.nn.sigmoid(gate) * up).astype(jnp.bfloat16)
        out = jnp.dot(act, wd_ref[0].astype(jnp.bfloat16),
                      preferred_element_type=jnp.float32)
        o_ref[...] = out


def _grouped_mlp(xs, w_gate, w_up, w_down, scalars):
    grid_spec = pltpu.PrefetchScalarGridSpec(
        num_scalar_prefetch=1,
        grid=(NB,),
        in_specs=[
            pl.BlockSpec((BLK, H), lambda i, be: (i, 0)),
            pl.BlockSpec((1, H, I), lambda i, be: (be[i, 0], 0, 0)),
            pl.BlockSpec((1, H, I), lambda i, be: (be[i, 0], 0, 0)),
            pl.BlockSpec((1, I, H), lambda i, be: (be[i, 0], 0, 0)),
        ],
        out_specs=pl.BlockSpec((BLK, H), lambda i, be: (i, 0)),
    )
    return pl.pallas_call(
        _mlp_block,
        grid_spec=grid_spec,
        out_shape=jax.ShapeDtypeStruct((NROWS, H), jnp.float32),
    )(scalars, xs, w_gate, w_up, w_down)


_NW = 32
_DISP_TPW = T // _NW
_COMB_TPW = T // _NW
_COMB_CH = _COMB_TPW // 4
_COMB_NCH = 4


def _sc_wid():
    return lax.axis_index("s") * 2 + lax.axis_index("c")


@functools.cache
def _sc_kernels():
    mesh = plsc.VectorSubcoreMesh(core_axis_name="c", subcore_axis_name="s")

    @functools.partial(
        pl.kernel, mesh=mesh,
        out_type=jax.ShapeDtypeStruct((NROWS, H), jnp.float32),
        scratch_types=[
            pltpu.VMEM((_DISP_TPW,), jnp.int32),
            pltpu.VMEM((_DISP_TPW,), jnp.int32),
            pltpu.VMEM((_DISP_TPW, H), jnp.float32),
            pltpu.SemaphoreType.DMA,
            pltpu.SemaphoreType.DMA,
            pltpu.SemaphoreType.DMA,
        ],
    )
    def _sc_dispatch(hidden_hbm, pos0_hbm, pos1_hbm, xs_hbm,
                     idx0, idx1, rows_v, sem0, sem1, sem2):
        tbase = _sc_wid() * _DISP_TPW
        ld0 = pltpu.async_copy(pos0_hbm.at[pl.ds(tbase, _DISP_TPW)], idx0,
                               sem0)
        ld1 = pltpu.async_copy(pos1_hbm.at[pl.ds(tbase, _DISP_TPW)], idx1,
                               sem1)
        ldh = pltpu.async_copy(hidden_hbm.at[pl.ds(tbase, _DISP_TPW)],
                               rows_v, sem2)
        ld0.wait()
        ld1.wait()
        ldh.wait()
        cp0 = pltpu.async_copy(rows_v, xs_hbm.at[idx0], sem0)
        cp1 = pltpu.async_copy(rows_v, xs_hbm.at[idx1], sem1)
        cp0.wait()
        cp1.wait()

    comb_set = [
        pltpu.VMEM((_COMB_CH,), jnp.int32),
        pltpu.VMEM((_COMB_CH,), jnp.int32),
        pltpu.VMEM((_COMB_CH, 16), jnp.float32),
        pltpu.VMEM((_COMB_CH, 16), jnp.float32),
        pltpu.VMEM((_COMB_CH, H), jnp.float32),
        pltpu.VMEM((_COMB_CH, H), jnp.float32),
        pltpu.SemaphoreType.DMA,
        pltpu.SemaphoreType.DMA,
    ]

    @functools.partial(
        pl.kernel, mesh=mesh,
        out_type=jax.ShapeDtypeStruct((T, H), jnp.float32),
        scratch_types=comb_set + comb_set,
    )
    def _sc_combine(ys_hbm, pos0_hbm, pos1_hbm, w0_hbm, w1_hbm, out_hbm,
                    *bufs):
        wbase = _sc_wid() * _COMB_TPW
        sets = [bufs[0:8], bufs[8:16]]

        def load_and_fire(c, st):
            i0, i1, wv0, wv1, b0, b1, s0, s1 = st
            tb = wbase + c * _COMB_CH
            pltpu.sync_copy(pos0_hbm.at[pl.ds(tb, _COMB_CH)], i0)
            pltpu.sync_copy(pos1_hbm.at[pl.ds(tb, _COMB_CH)], i1)
            pltpu.sync_copy(w0_hbm.at[pl.ds(tb, _COMB_CH)], wv0)
            pltpu.sync_copy(w1_hbm.at[pl.ds(tb, _COMB_CH)], wv1)
            return (pltpu.async_copy(ys_hbm.at[i0], b0, s0),
                    pltpu.async_copy(ys_hbm.at[i1], b1, s1))

        cps = load_and_fire(0, sets[0])
        for c in range(_COMB_NCH):
            nxt_cps = (load_and_fire(c + 1, sets[(c + 1) % 2])
                       if c + 1 < _COMB_NCH else None)
            cps[0].wait()
            cps[1].wait()
            _, _, wv0, wv1, b0, b1, _, _ = sets[c % 2]

            def _add(r, _, wv0=wv0, wv1=wv1, b0=b0, b1=b1):
                a = wv0[r, :]
                b = wv1[r, :]
                for u in range(H // 16):
                    sl = pl.ds(u * 16, 16)
                    b0[r, sl] = b0[r, sl] * a + b1[r, sl] * b
                return 0

            lax.fori_loop(0, _COMB_CH, _add, 0, unroll=False)
            pltpu.sync_copy(b0, out_hbm.at[pl.ds(wbase + c * _COMB_CH,
                                                 _COMB_CH)])
            cps = nxt_cps

    return _sc_dispatch, _sc_combine


def kernel(hidden_states, expert_affinities, expert_index, w_gate, w_up,
           w_down, seq_len):
    sc_dispatch, sc_combine = _sc_kernels()
    pos0, pos1, w0x, w1x, sc2 = _routing(expert_affinities, expert_index)
    pos0, pos1 = pos0.reshape(T), pos1.reshape(T)
    xs = sc_dispatch(hidden_states, pos0, pos1)
    ys = _grouped_mlp(xs, w_gate, w_up, w_down, sc2)
    out = sc_combine(ys, pos0, pos1, w0x, w1x)
    return out
